# baseline (speedup 1.0000x reference)
import functools
import jax
import jax.numpy as jnp
from jax import lax
from jax.experimental import pallas as pl
from jax.experimental.pallas import tpu as pltpu
from jax.experimental.pallas import tpu_sc as plsc

NR, ED, B = 6, 512, 4096
NC, NS = 2, 16
NW = NC * NS
BPW = B // NW
RC = 16
NOUT = BPW // RC
IDXP = 16
LANES = 16
VPR = ED // LANES

_mesh = plsc.VectorSubcoreMesh(core_axis_name="c", subcore_axis_name="s")


@functools.partial(
    pl.kernel, mesh=_mesh,
    out_type=jax.ShapeDtypeStruct((B, NR, ED), jnp.float32),
    scratch_types=[
        pltpu.VMEM((IDXP,), jnp.int32),
        pltpu.VMEM((IDXP, ED), jnp.float32),
        pltpu.VMEM((RC, NR, ED), jnp.float32),
        pltpu.SemaphoreType.DMA,
        pltpu.SemaphoreType.DMA((NOUT,)),
    ],
)
def _sc_broadcast(table_hbm, idx_hbm, out_hbm, idx_v, rows_v, buf_v,
                  gsem, osems):
    wid = lax.axis_index("s") * NC + lax.axis_index("c")
    base = wid * BPW
    pltpu.sync_copy(idx_hbm, idx_v)
    pltpu.async_copy(table_hbm.at[idx_v], rows_v, gsem).wait()
    for r in range(NR):
        vals = [rows_v[r, pl.ds(i * LANES, LANES)] for i in range(VPR)]
        for c in range(RC):
            for i in range(VPR):
                buf_v[c, r, pl.ds(i * LANES, LANES)] = vals[i]
    for k in range(NOUT):
        pltpu.make_async_copy(
            buf_v, out_hbm.at[pl.ds(base + k * RC, RC)], osems.at[k]).start()
    for k in range(NOUT):
        pltpu.make_async_copy(
            buf_v, out_hbm.at[pl.ds(base + k * RC, RC)], osems.at[k]).wait()


def kernel(token_embed_weight, region_ids, batch_size):
    del batch_size
    ids = region_ids.astype(jnp.int32)
    idx_pad = jnp.concatenate([ids, jnp.zeros((IDXP - NR,), jnp.int32)])
    return _sc_broadcast(token_embed_weight, idx_pad)


# speedup vs baseline: 1.2385x; 1.2385x over previous
import functools
import jax
import jax.numpy as jnp
from jax import lax
from jax.experimental import pallas as pl
from jax.experimental.pallas import tpu as pltpu
from jax.experimental.pallas import tpu_sc as plsc

NR, ED, B = 6, 512, 4096
NC, NS = 2, 16
NW = NC * NS
BPW = B // NW
RC = 16
NOUT = BPW // RC
SROWS = 48

_mesh = plsc.VectorSubcoreMesh(core_axis_name="c", subcore_axis_name="s")


@functools.partial(
    pl.kernel, mesh=_mesh,
    out_type=(
        jax.ShapeDtypeStruct((B, NR, ED), jnp.float32),
        jax.ShapeDtypeStruct((NW, NR, ED), jnp.float32),
    ),
    scratch_types=[
        pltpu.VMEM((SROWS,), jnp.int32),
        pltpu.VMEM((SROWS, ED), jnp.float32),
        pltpu.VMEM((RC, NR, ED), jnp.float32),
        pltpu.SemaphoreType.DMA,
        pltpu.SemaphoreType.DMA((RC,)),
        pltpu.SemaphoreType.DMA((NOUT,)),
    ],
)
def _sc_broadcast(table_hbm, idx_hbm, out_hbm, stage_hbm, idx_v, rows_v,
                  buf_v, gsem, fsems, osems):
    wid = lax.axis_index("s") * NC + lax.axis_index("c")
    base = wid * BPW
    pltpu.sync_copy(idx_hbm, idx_v)
    pltpu.async_copy(table_hbm.at[idx_v], rows_v, gsem).wait()
    pltpu.make_async_copy(
        rows_v.at[pl.ds(0, NR)], stage_hbm.at[wid], gsem).start()
    pltpu.make_async_copy(
        rows_v.at[pl.ds(0, NR)], stage_hbm.at[wid], gsem).wait()
    for c in range(RC):
        pltpu.make_async_copy(
            stage_hbm.at[wid], buf_v.at[c], fsems.at[c]).start()
    for c in range(RC):
        pltpu.make_async_copy(
            stage_hbm.at[wid], buf_v.at[c], fsems.at[c]).wait()
    for k in range(NOUT):
        pltpu.make_async_copy(
            buf_v, out_hbm.at[pl.ds(base + k * RC, RC)], osems.at[k]).start()
    for k in range(NOUT):
        pltpu.make_async_copy(
            buf_v, out_hbm.at[pl.ds(base + k * RC, RC)], osems.at[k]).wait()


def kernel(token_embed_weight, region_ids, batch_size):
    del batch_size
    idx_rep = jnp.tile(region_ids.astype(jnp.int32), SROWS // NR)
    out, _ = _sc_broadcast(token_embed_weight, idx_rep)
    return out


# speedup vs baseline: 1.5510x; 1.2523x over previous
import functools
import jax
import jax.numpy as jnp
from jax import lax
from jax.experimental import pallas as pl
from jax.experimental.pallas import tpu as pltpu
from jax.experimental.pallas import tpu_sc as plsc

NR, ED, B = 6, 512, 4096
NC, NS = 2, 16
SROWS = 48
BLK = 512

_mesh = plsc.VectorSubcoreMesh(core_axis_name="c", subcore_axis_name="s")


@functools.partial(
    pl.kernel, mesh=_mesh,
    out_type=jax.ShapeDtypeStruct((NR, ED), jnp.float32),
    scratch_types=[
        pltpu.VMEM((SROWS,), jnp.int32),
        pltpu.VMEM((SROWS, ED), jnp.float32),
        pltpu.SemaphoreType.DMA,
    ],
)
def _sc_gather(table_hbm, idx_hbm, out_hbm, idx_v, rows_v, gsem):
    wid = lax.axis_index("s") * NC + lax.axis_index("c")

    @pl.when(wid == 0)
    def _():
        pltpu.sync_copy(idx_hbm, idx_v)
        pltpu.async_copy(table_hbm.at[idx_v], rows_v, gsem).wait()
        pltpu.make_async_copy(
            rows_v.at[pl.ds(0, NR)], out_hbm, gsem).start()
        pltpu.make_async_copy(
            rows_v.at[pl.ds(0, NR)], out_hbm, gsem).wait()


def _tc_body(t_ref, out_ref):
    out_ref[...] = jnp.broadcast_to(t_ref[...][None], (BLK, NR, ED))


def kernel(token_embed_weight, region_ids, batch_size):
    del batch_size
    idx_rep = jnp.tile(region_ids.astype(jnp.int32), SROWS // NR)
    tokens = _sc_gather(token_embed_weight, idx_rep)
    return pl.pallas_call(
        _tc_body,
        grid=(B // BLK,),
        in_specs=[pl.BlockSpec((NR, ED), lambda i: (0, 0))],
        out_specs=pl.BlockSpec((BLK, NR, ED), lambda i: (i, 0, 0)),
        out_shape=jax.ShapeDtypeStruct((B, NR, ED), jnp.float32),
    )(tokens)
